# baseline (speedup 1.0000x reference)
import functools

import jax
import jax.numpy as jnp
from jax import lax
from jax.experimental import pallas as pl
from jax.experimental.pallas import tpu as pltpu
from jax.experimental.pallas import tpu_sc as plsc

N_NODES = 10000
N_EDGES = 320000
F_IN = 16
NH = 4
CH = 128
ET = N_EDGES + N_NODES
ET_PAD = 331776
REG = 10240
GW = 128
SW = 128
BB = 2048
DB = 1000
KL = 128 // F_IN
PB = BB // KL
PROWS = ET_PAD // KL

_vector_mesh = plsc.VectorSubcoreMesh(core_axis_name="core",
                                      subcore_axis_name="subcore")
_sc_untiled = pltpu.CompilerParams(use_tc_tiling_on_sc=False)


CH_BLOCKS = (12, 27, 27, 27, 27, 27, 15)
NCH = len(CH_BLOCKS)


def _gather_call(feat, idx_cat, ne):
    nidx = 2 * ne

    @functools.partial(
        pl.kernel,
        out_type=jax.ShapeDtypeStruct((nidx, F_IN), jnp.float32),
        mesh=_vector_mesh,
        compiler_params=_sc_untiled,
    )
    def gather_k(feat_hbm, i_hbm, o_hbm):
        def body(i_vmem, o_vmem):
            pltpu.sync_copy(feat_hbm.at[i_vmem.at[0]], o_vmem)

        pltpu.emit_pipeline(
            body,
            grid=(nidx // GW,),
            in_specs=[pl.BlockSpec((1, GW), lambda i: (i, 0))],
            out_specs=[pl.BlockSpec((GW, F_IN), lambda i: (i, 0))],
            core_axis_name=("core", "subcore"),
            dimension_semantics=(pltpu.PARALLEL,),
        )(i_hbm, o_hbm)

    return gather_k(feat, idx_cat.reshape(nidx // GW, GW)).reshape(
        2 * ne // KL, 128)


def _logit_body(gs_ref, gd_ref, wlrb_ref, b2b_ref, att2b_ref,
                r4b_ref, pmatb_ref, cvecb_ref, pay_ref):
    ps = gs_ref[...]
    pd = gd_ref[...]
    pc = jnp.concatenate([ps, pd], axis=1).astype(jnp.bfloat16)
    u = jnp.dot(pc, wlrb_ref[...], preferred_element_type=jnp.float32)
    u = u + b2b_ref[...]
    e = jnp.where(u >= 0.0, u, 0.2 * u)
    logits = jnp.dot(e.astype(jnp.bfloat16), att2b_ref[...],
                     preferred_element_type=jnp.float32)
    a = jnp.exp(logits)
    m = jnp.dot(a, r4b_ref[...], preferred_element_type=jnp.float32)
    g4 = jnp.dot(ps, pmatb_ref[...],
                 preferred_element_type=jnp.float32) + cvecb_ref[...]
    pay_ref[...] = m * g4


def _logit_call(g2, wlrb, b2b, att2b, r4b, pmatb, cvecb, ne):
    nb = ne // BB
    return pl.pallas_call(
        _logit_body,
        grid=(nb,),
        in_specs=[
            pl.BlockSpec((PB, 128), lambda i: (i, 0)),
            pl.BlockSpec((PB, 128), lambda i: (i + nb, 0)),
            pl.BlockSpec((256, KL * 512), lambda i: (0, 0)),
            pl.BlockSpec((1, KL * 512), lambda i: (0, 0)),
            pl.BlockSpec((KL * 512, KL * NH), lambda i: (0, 0)),
            pl.BlockSpec((KL * NH, KL * CH), lambda i: (0, 0)),
            pl.BlockSpec((128, KL * CH), lambda i: (0, 0)),
            pl.BlockSpec((1, KL * CH), lambda i: (0, 0)),
        ],
        out_specs=pl.BlockSpec((PB, KL * CH), lambda i: (i, 0)),
        out_shape=jax.ShapeDtypeStruct((ne // KL, KL * CH), jnp.float32),
    )(g2, g2, wlrb, b2b, att2b, r4b, pmatb, cvecb)


def _scatter_call(pay, idx_perm, zrows, ne):
    steps_r = ne // KL // SW
    slc = REG // 16

    @functools.partial(
        pl.kernel,
        out_type=jax.ShapeDtypeStruct((2, REG, CH), jnp.float32),
        mesh=_vector_mesh,
        scratch_types=[pltpu.VMEM_SHARED((REG, CH), jnp.float32)],
    )
    def scatter_k(pay_hbm, i_hbm, z_hbm, gp_hbm, table):
        c = lax.axis_index("core")
        s = lax.axis_index("subcore")
        pltpu.sync_copy(z_hbm, table.at[pl.ds(s * slc, slc)])
        plsc.subcore_barrier()

        def body(pay_vmem, i_vmem):
            pltpu.sync_copy(pay_vmem, table.at[i_vmem.at[0, 0]], add=True)

        pltpu.emit_pipeline(
            body,
            grid=(steps_r, KL),
            in_specs=[
                pl.BlockSpec((SW, 128), lambda i, k: (i, k)),
                pl.BlockSpec((1, 1, SW), lambda i, k: (i, k, 0)),
            ],
            out_specs=[],
            core_axis_name=("core", "subcore"),
            dimension_semantics=(pltpu.PARALLEL, pltpu.PARALLEL),
        )(pay_hbm, i_hbm)
        plsc.subcore_barrier()
        pltpu.sync_copy(table.at[pl.ds(s * slc, slc)],
                        gp_hbm.at[c, pl.ds(s * slc, slc)])

    return scatter_k(pay, idx_perm, zrows)


def _final_body(*refs):
    gp_refs = refs[:NCH]
    wl_ref, bl_ref, bias_ref, o_ref = refs[NCH:]
    g = gp_refs[0][0] + gp_refs[0][1]
    for r in gp_refs[1:]:
        g = g + r[0] + r[1]
    outs = []
    for h in range(NH):
        feat_sum = g[:, 32 * h:32 * h + F_IN]
        ssum = g[:, 32 * h + F_IN:32 * h + F_IN + 1]
        y = jnp.dot(feat_sum, wl_ref[:, h * CH:(h + 1) * CH],
                    preferred_element_type=jnp.float32)
        rr = 1.0 / (ssum + 1e-16)
        outs.append(y * rr + bl_ref[:, h * CH:(h + 1) * CH] * (ssum * rr)
                    + bias_ref[:, h * CH:(h + 1) * CH])
    o_ref[...] = jnp.concatenate(outs, axis=1)


def _final_call(gpairs, W_l, bl2, bias2):
    gspec = pl.BlockSpec((2, DB, CH), lambda d: (0, d, 0))
    return pl.pallas_call(
        _final_body,
        grid=(N_NODES // DB,),
        in_specs=[gspec] * NCH + [
            pl.BlockSpec((F_IN, NH * CH), lambda d: (0, 0)),
            pl.BlockSpec((1, NH * CH), lambda d: (0, 0)),
            pl.BlockSpec((1, NH * CH), lambda d: (0, 0)),
        ],
        out_specs=pl.BlockSpec((DB, NH * CH), lambda d: (d, 0)),
        out_shape=jax.ShapeDtypeStruct((N_NODES, NH * CH), jnp.float32),
    )(*gpairs, W_l, bl2, bias2)


def kernel(edge_feat, edge_to_edge_index, W_l, b_l, W_r, b_r, att, bias):
    loop = jnp.arange(N_NODES, dtype=jnp.int32)
    pad = ET_PAD - ET
    src_all = jnp.concatenate(
        [edge_to_edge_index[0], loop, jnp.zeros((pad,), jnp.int32)])
    dst_real = jnp.concatenate([edge_to_edge_index[1], loop])
    dst_g = jnp.concatenate([dst_real, jnp.zeros((pad,), jnp.int32)])
    idx_dst = jnp.concatenate(
        [dst_real, jnp.full((pad,), N_NODES, jnp.int32)])
    idx_perm = idx_dst.reshape(PROWS // SW, SW, KL).transpose(0, 2, 1)
    b2 = (b_l + b_r).reshape(1, NH * CH)
    eye4 = jnp.eye(NH, dtype=jnp.float32)
    eye8 = jnp.eye(KL, dtype=jnp.float32)
    att2 = (eye4[:, None, :] * att[:, :, None]).reshape(NH * CH, NH)
    lane = jnp.arange(CH)
    r4 = (eye4[:, lane // 32] * (lane % 32 < 17)[None, :]).astype(jnp.float32)
    pmat = ((lane % 32)[None, :] == jnp.arange(F_IN)[:, None]).astype(
        jnp.float32)
    cvec = ((lane % 32) == F_IN).astype(jnp.float32).reshape(1, CH)
    wlrb = jnp.concatenate(
        [jnp.kron(eye8, W_l), jnp.kron(eye8, W_r)], axis=0
    ).astype(jnp.bfloat16)
    b2b = jnp.tile(b2, (1, KL))
    att2b = jnp.kron(eye8, att2).astype(jnp.bfloat16)
    r4b = jnp.kron(eye8, r4)
    pmatb = jnp.kron(eye8, pmat)
    cvecb = jnp.tile(cvec, (1, KL))
    zrows = jnp.zeros((REG // 16, CH), jnp.float32)

    gpairs = []
    e0 = 0
    for c in range(NCH):
        ne = CH_BLOCKS[c] * BB
        idx_cat_c = jnp.concatenate([src_all[e0:e0 + ne],
                                     dst_g[e0:e0 + ne]])
        g2c = _gather_call(edge_feat, idx_cat_c, ne)
        payc = _logit_call(g2c, wlrb, b2b, att2b, r4b, pmatb, cvecb, ne)
        s0 = e0 // KL // SW
        gpairs.append(_scatter_call(
            payc, idx_perm[s0:s0 + ne // KL // SW], zrows, ne))
        e0 += ne
    return _final_call(gpairs, W_l, b_l.reshape(1, NH * CH),
                       bias.reshape(1, NH * CH))


# speedup vs baseline: 1.0200x; 1.0200x over previous
import functools

import jax
import jax.numpy as jnp
from jax import lax
from jax.experimental import pallas as pl
from jax.experimental.pallas import tpu as pltpu
from jax.experimental.pallas import tpu_sc as plsc

N_NODES = 10000
N_EDGES = 320000
F_IN = 16
NH = 4
CH = 128
ET = N_EDGES + N_NODES
ET_PAD = 331776
REG = 10240
GW = 128
SW = 128
BB = 2048
DB = 1000
KL = 128 // F_IN
PB = BB // KL
PROWS = ET_PAD // KL

_vector_mesh = plsc.VectorSubcoreMesh(core_axis_name="core",
                                      subcore_axis_name="subcore")
_sc_untiled = pltpu.CompilerParams(use_tc_tiling_on_sc=False)


CH_BLOCKS = (27, 27, 27, 27, 27, 27)
NCH = len(CH_BLOCKS)


def _gather_call(feat, idx_cat, ne):
    nidx = 2 * ne

    @functools.partial(
        pl.kernel,
        out_type=jax.ShapeDtypeStruct((nidx, F_IN), jnp.float32),
        mesh=_vector_mesh,
        compiler_params=_sc_untiled,
    )
    def gather_k(feat_hbm, i_hbm, o_hbm):
        def body(i_vmem, o_vmem):
            pltpu.sync_copy(feat_hbm.at[i_vmem.at[0]], o_vmem)

        pltpu.emit_pipeline(
            body,
            grid=(nidx // GW,),
            in_specs=[pl.BlockSpec((1, GW), lambda i: (i, 0))],
            out_specs=[pl.BlockSpec((GW, F_IN), lambda i: (i, 0))],
            core_axis_name=("core", "subcore"),
            dimension_semantics=(pltpu.PARALLEL,),
        )(i_hbm, o_hbm)

    return gather_k(feat, idx_cat.reshape(nidx // GW, GW)).reshape(
        2 * ne // KL, 128)


def _logit_body(gs_ref, gd_ref, wlrb_ref, b2b_ref, att2b_ref,
                r4b_ref, pmatb_ref, cvecb_ref, pay_ref):
    ps = gs_ref[...]
    pd = gd_ref[...]
    pc = jnp.concatenate([ps, pd], axis=1).astype(jnp.bfloat16)
    u = jnp.dot(pc, wlrb_ref[...], preferred_element_type=jnp.float32)
    u = u + b2b_ref[...]
    e = jnp.where(u >= 0.0, u, 0.2 * u)
    logits = jnp.dot(e.astype(jnp.bfloat16), att2b_ref[...],
                     preferred_element_type=jnp.float32)
    a = jnp.exp(logits)
    m = jnp.dot(a, r4b_ref[...], preferred_element_type=jnp.float32)
    g4 = jnp.dot(ps, pmatb_ref[...],
                 preferred_element_type=jnp.float32) + cvecb_ref[...]
    pay_ref[...] = m * g4


def _logit_call(g2, wlrb, b2b, att2b, r4b, pmatb, cvecb, ne):
    nb = ne // BB
    return pl.pallas_call(
        _logit_body,
        grid=(nb,),
        in_specs=[
            pl.BlockSpec((PB, 128), lambda i: (i, 0)),
            pl.BlockSpec((PB, 128), lambda i: (i + nb, 0)),
            pl.BlockSpec((256, KL * 512), lambda i: (0, 0)),
            pl.BlockSpec((1, KL * 512), lambda i: (0, 0)),
            pl.BlockSpec((KL * 512, KL * NH), lambda i: (0, 0)),
            pl.BlockSpec((KL * NH, KL * CH), lambda i: (0, 0)),
            pl.BlockSpec((128, KL * CH), lambda i: (0, 0)),
            pl.BlockSpec((1, KL * CH), lambda i: (0, 0)),
        ],
        out_specs=pl.BlockSpec((PB, KL * CH), lambda i: (i, 0)),
        out_shape=jax.ShapeDtypeStruct((ne // KL, KL * CH), jnp.float32),
    )(g2, g2, wlrb, b2b, att2b, r4b, pmatb, cvecb)


def _scatter_call(pay, idx_perm, init_pair, ne):
    steps_r = ne // KL // SW
    slc = REG // 16

    @functools.partial(
        pl.kernel,
        out_type=jax.ShapeDtypeStruct((2, REG, CH), jnp.float32),
        mesh=_vector_mesh,
        scratch_types=[pltpu.VMEM_SHARED((REG, CH), jnp.float32)],
    )
    def scatter_k(pay_hbm, i_hbm, init_hbm, gp_hbm, table):
        c = lax.axis_index("core")
        s = lax.axis_index("subcore")
        pltpu.sync_copy(init_hbm.at[c, pl.ds(s * slc, slc)],
                        table.at[pl.ds(s * slc, slc)])
        plsc.subcore_barrier()

        def body(pay_vmem, i_vmem):
            pltpu.sync_copy(pay_vmem, table.at[i_vmem.at[0, 0]], add=True)

        pltpu.emit_pipeline(
            body,
            grid=(steps_r, KL),
            in_specs=[
                pl.BlockSpec((SW, 128), lambda i, k: (i, k)),
                pl.BlockSpec((1, 1, SW), lambda i, k: (i, k, 0)),
            ],
            out_specs=[],
            core_axis_name=("core", "subcore"),
            dimension_semantics=(pltpu.PARALLEL, pltpu.PARALLEL),
        )(pay_hbm, i_hbm)
        plsc.subcore_barrier()
        pltpu.sync_copy(table.at[pl.ds(s * slc, slc)],
                        gp_hbm.at[c, pl.ds(s * slc, slc)])

    return scatter_k(pay, idx_perm, init_pair)


def _final_body(g_ref, wl_ref, bl_ref, bias_ref, o_ref):
    g = g_ref[0] + g_ref[1]
    outs = []
    for h in range(NH):
        feat_sum = g[:, 32 * h:32 * h + F_IN]
        ssum = g[:, 32 * h + F_IN:32 * h + F_IN + 1]
        y = jnp.dot(feat_sum, wl_ref[:, h * CH:(h + 1) * CH],
                    preferred_element_type=jnp.float32)
        rr = 1.0 / (ssum + 1e-16)
        outs.append(y * rr + bl_ref[:, h * CH:(h + 1) * CH] * (ssum * rr)
                    + bias_ref[:, h * CH:(h + 1) * CH])
    o_ref[...] = jnp.concatenate(outs, axis=1)


def _final_call(gpair, W_l, bl2, bias2):
    return pl.pallas_call(
        _final_body,
        grid=(N_NODES // DB,),
        in_specs=[
            pl.BlockSpec((2, DB, CH), lambda d: (0, d, 0)),
            pl.BlockSpec((F_IN, NH * CH), lambda d: (0, 0)),
            pl.BlockSpec((1, NH * CH), lambda d: (0, 0)),
            pl.BlockSpec((1, NH * CH), lambda d: (0, 0)),
        ],
        out_specs=pl.BlockSpec((DB, NH * CH), lambda d: (d, 0)),
        out_shape=jax.ShapeDtypeStruct((N_NODES, NH * CH), jnp.float32),
    )(gpair, W_l, bl2, bias2)


def kernel(edge_feat, edge_to_edge_index, W_l, b_l, W_r, b_r, att, bias):
    loop = jnp.arange(N_NODES, dtype=jnp.int32)
    pad = ET_PAD - ET
    src_all = jnp.concatenate(
        [edge_to_edge_index[0], loop, jnp.zeros((pad,), jnp.int32)])
    dst_real = jnp.concatenate([edge_to_edge_index[1], loop])
    dst_g = jnp.concatenate([dst_real, jnp.zeros((pad,), jnp.int32)])
    idx_dst = jnp.concatenate(
        [dst_real, jnp.full((pad,), N_NODES, jnp.int32)])
    idx_perm = idx_dst.reshape(PROWS // SW, SW, KL).transpose(0, 2, 1)
    b2 = (b_l + b_r).reshape(1, NH * CH)
    eye4 = jnp.eye(NH, dtype=jnp.float32)
    eye8 = jnp.eye(KL, dtype=jnp.float32)
    att2 = (eye4[:, None, :] * att[:, :, None]).reshape(NH * CH, NH)
    lane = jnp.arange(CH)
    r4 = (eye4[:, lane // 32] * (lane % 32 < 17)[None, :]).astype(jnp.float32)
    pmat = ((lane % 32)[None, :] == jnp.arange(F_IN)[:, None]).astype(
        jnp.float32)
    cvec = ((lane % 32) == F_IN).astype(jnp.float32).reshape(1, CH)
    wlrb = jnp.concatenate(
        [jnp.kron(eye8, W_l), jnp.kron(eye8, W_r)], axis=0
    ).astype(jnp.bfloat16)
    b2b = jnp.tile(b2, (1, KL))
    att2b = jnp.kron(eye8, att2).astype(jnp.bfloat16)
    r4b = jnp.kron(eye8, r4)
    pmatb = jnp.kron(eye8, pmat)
    cvecb = jnp.tile(cvec, (1, KL))
    gpair = jnp.zeros((2, REG, CH), jnp.float32)
    e0 = 0
    for c in range(NCH):
        ne = CH_BLOCKS[c] * BB
        idx_cat_c = jnp.concatenate([src_all[e0:e0 + ne],
                                     dst_g[e0:e0 + ne]])
        g2c = _gather_call(edge_feat, idx_cat_c, ne)
        payc = _logit_call(g2c, wlrb, b2b, att2b, r4b, pmatb, cvecb, ne)
        s0 = e0 // KL // SW
        gpair = _scatter_call(
            payc, idx_perm[s0:s0 + ne // KL // SW], gpair, ne)
        e0 += ne
    return _final_call(gpair, W_l, b_l.reshape(1, NH * CH),
                       bias.reshape(1, NH * CH))


# speedup vs baseline: 1.0561x; 1.0354x over previous
import functools

import jax
import jax.numpy as jnp
from jax import lax
from jax.experimental import pallas as pl
from jax.experimental.pallas import tpu as pltpu
from jax.experimental.pallas import tpu_sc as plsc

N_NODES = 10000
N_EDGES = 320000
F_IN = 16
NH = 4
CH = 128
ET = N_EDGES + N_NODES
ET_PAD = 331776
REG = 10240
GW = 128
SW = 128
BB = 2048
DB = 1000
KL = 128 // F_IN
PB = BB // KL
PROWS = ET_PAD // KL

_vector_mesh = plsc.VectorSubcoreMesh(core_axis_name="core",
                                      subcore_axis_name="subcore")
_sc_untiled = pltpu.CompilerParams(use_tc_tiling_on_sc=False)


CH_BLOCKS = (27, 27, 27, 27, 27, 27)
NCH = len(CH_BLOCKS)


def _gather_call(feat, idx_cat, ne):
    nidx = 2 * ne

    @functools.partial(
        pl.kernel,
        out_type=jax.ShapeDtypeStruct((nidx, F_IN), jnp.float32),
        mesh=_vector_mesh,
        compiler_params=_sc_untiled,
    )
    def gather_k(feat_hbm, i_hbm, o_hbm):
        def body(i_vmem, o_vmem):
            pltpu.sync_copy(feat_hbm.at[i_vmem.at[0]], o_vmem)

        pltpu.emit_pipeline(
            body,
            grid=(nidx // GW,),
            in_specs=[pl.BlockSpec((1, GW), lambda i: (i, 0))],
            out_specs=[pl.BlockSpec((GW, F_IN), lambda i: (i, 0))],
            core_axis_name=("core", "subcore"),
            dimension_semantics=(pltpu.PARALLEL,),
        )(i_hbm, o_hbm)

    return gather_k(feat, idx_cat.reshape(nidx // GW, GW)).reshape(
        2 * ne // KL, 128)


def _logit_body(gs_ref, gd_ref, wlrb_ref, b2b_ref, att2b_ref,
                r4b_ref, pmatb_ref, cvecb_ref, pay_ref):
    ps = gs_ref[...]
    pd = gd_ref[...]
    pc = jnp.concatenate([ps, pd], axis=1).astype(jnp.bfloat16)
    u = jnp.dot(pc, wlrb_ref[...], preferred_element_type=jnp.float32)
    u = u + b2b_ref[...]
    e = jnp.where(u >= 0.0, u, 0.2 * u)
    logits = jnp.dot(e.astype(jnp.bfloat16), att2b_ref[...],
                     preferred_element_type=jnp.float32)
    a = jnp.exp(logits)
    m = jnp.dot(a, r4b_ref[...], preferred_element_type=jnp.float32)
    g4 = jnp.dot(ps, pmatb_ref[...],
                 preferred_element_type=jnp.float32) + cvecb_ref[...]
    pay_ref[...] = m * g4


def _logit_call(g2, wlrb, b2b, att2b, r4b, pmatb, cvecb, ne):
    nb = ne // BB
    return pl.pallas_call(
        _logit_body,
        grid=(nb,),
        in_specs=[
            pl.BlockSpec((PB, 128), lambda i: (i, 0)),
            pl.BlockSpec((PB, 128), lambda i: (i + nb, 0)),
            pl.BlockSpec((256, KL * 512), lambda i: (0, 0)),
            pl.BlockSpec((1, KL * 512), lambda i: (0, 0)),
            pl.BlockSpec((KL * 512, KL * NH), lambda i: (0, 0)),
            pl.BlockSpec((KL * NH, KL * CH), lambda i: (0, 0)),
            pl.BlockSpec((128, KL * CH), lambda i: (0, 0)),
            pl.BlockSpec((1, KL * CH), lambda i: (0, 0)),
        ],
        out_specs=pl.BlockSpec((PB, KL * CH), lambda i: (i, 0)),
        out_shape=jax.ShapeDtypeStruct((ne // KL, KL * CH), jnp.float32),
    )(g2, g2, wlrb, b2b, att2b, r4b, pmatb, cvecb)


def _scatter_call(pay, idx_perm, zrows, ne):
    steps_r = ne // KL // SW
    slc = REG // 16

    @functools.partial(
        pl.kernel,
        out_type=jax.ShapeDtypeStruct((2, REG, CH), jnp.float32),
        mesh=_vector_mesh,
        scratch_types=[pltpu.VMEM_SHARED((REG, CH), jnp.float32)],
    )
    def scatter_k(pay_hbm, i_hbm, z_hbm, gp_hbm, table):
        c = lax.axis_index("core")
        s = lax.axis_index("subcore")
        pltpu.sync_copy(z_hbm, table.at[pl.ds(s * slc, slc)])
        plsc.subcore_barrier()

        def body(pay_vmem, i_vmem):
            pltpu.sync_copy(pay_vmem, table.at[i_vmem.at[0, 0]], add=True)

        pltpu.emit_pipeline(
            body,
            grid=(steps_r, KL),
            in_specs=[
                pl.BlockSpec((SW, 128), lambda i, k: (i, k)),
                pl.BlockSpec((1, 1, SW), lambda i, k: (i, k, 0)),
            ],
            out_specs=[],
            core_axis_name=("core", "subcore"),
            dimension_semantics=(pltpu.PARALLEL, pltpu.PARALLEL),
        )(pay_hbm, i_hbm)
        plsc.subcore_barrier()
        pltpu.sync_copy(table.at[pl.ds(s * slc, slc)],
                        gp_hbm.at[c, pl.ds(s * slc, slc)])

    return scatter_k(pay, idx_perm, zrows)


def _final_body(*refs):
    gp_refs = refs[:NCH]
    wl_ref, bl_ref, bias_ref, o_ref = refs[NCH:]
    g = gp_refs[0][0] + gp_refs[0][1]
    for r in gp_refs[1:]:
        g = g + r[0] + r[1]
    outs = []
    for h in range(NH):
        feat_sum = g[:, 32 * h:32 * h + F_IN]
        ssum = g[:, 32 * h + F_IN:32 * h + F_IN + 1]
        y = jnp.dot(feat_sum, wl_ref[:, h * CH:(h + 1) * CH],
                    preferred_element_type=jnp.float32)
        rr = 1.0 / (ssum + 1e-16)
        outs.append(y * rr + bl_ref[:, h * CH:(h + 1) * CH] * (ssum * rr)
                    + bias_ref[:, h * CH:(h + 1) * CH])
    o_ref[...] = jnp.concatenate(outs, axis=1)


def _final_call(gpairs, W_l, bl2, bias2):
    gspec = pl.BlockSpec((2, DB, CH), lambda d: (0, d, 0))
    return pl.pallas_call(
        _final_body,
        grid=(N_NODES // DB,),
        in_specs=[gspec] * NCH + [
            pl.BlockSpec((F_IN, NH * CH), lambda d: (0, 0)),
            pl.BlockSpec((1, NH * CH), lambda d: (0, 0)),
            pl.BlockSpec((1, NH * CH), lambda d: (0, 0)),
        ],
        out_specs=pl.BlockSpec((DB, NH * CH), lambda d: (d, 0)),
        out_shape=jax.ShapeDtypeStruct((N_NODES, NH * CH), jnp.float32),
    )(*gpairs, W_l, bl2, bias2)


def kernel(edge_feat, edge_to_edge_index, W_l, b_l, W_r, b_r, att, bias):
    loop = jnp.arange(N_NODES, dtype=jnp.int32)
    pad = ET_PAD - ET
    src_all = jnp.concatenate(
        [edge_to_edge_index[0], loop, jnp.zeros((pad,), jnp.int32)])
    dst_real = jnp.concatenate([edge_to_edge_index[1], loop])
    dst_g = jnp.concatenate([dst_real, jnp.zeros((pad,), jnp.int32)])
    idx_dst = jnp.concatenate(
        [dst_real, jnp.full((pad,), N_NODES, jnp.int32)])
    idx_perm = idx_dst.reshape(PROWS // SW, SW, KL).transpose(0, 2, 1)
    b2 = (b_l + b_r).reshape(1, NH * CH)
    eye4 = jnp.eye(NH, dtype=jnp.float32)
    eye8 = jnp.eye(KL, dtype=jnp.float32)
    att2 = (eye4[:, None, :] * att[:, :, None]).reshape(NH * CH, NH)
    lane = jnp.arange(CH)
    r4 = (eye4[:, lane // 32] * (lane % 32 < 17)[None, :]).astype(jnp.float32)
    pmat = ((lane % 32)[None, :] == jnp.arange(F_IN)[:, None]).astype(
        jnp.float32)
    cvec = ((lane % 32) == F_IN).astype(jnp.float32).reshape(1, CH)
    wlrb = jnp.concatenate(
        [jnp.kron(eye8, W_l), jnp.kron(eye8, W_r)], axis=0
    ).astype(jnp.bfloat16)
    b2b = jnp.tile(b2, (1, KL))
    att2b = jnp.kron(eye8, att2).astype(jnp.bfloat16)
    r4b = jnp.kron(eye8, r4)
    pmatb = jnp.kron(eye8, pmat)
    cvecb = jnp.tile(cvec, (1, KL))
    zrows = jnp.zeros((REG // 16, CH), jnp.float32)

    gpairs = []
    e0 = 0
    for c in range(NCH):
        ne = CH_BLOCKS[c] * BB
        idx_cat_c = jnp.concatenate([src_all[e0:e0 + ne],
                                     dst_g[e0:e0 + ne]])
        g2c = _gather_call(edge_feat, idx_cat_c, ne)
        payc = _logit_call(g2c, wlrb, b2b, att2b, r4b, pmatb, cvecb, ne)
        s0 = e0 // KL // SW
        gpairs.append(_scatter_call(
            payc, idx_perm[s0:s0 + ne // KL // SW], zrows, ne))
        e0 += ne
    return _final_call(gpairs, W_l, b_l.reshape(1, NH * CH),
                       bias.reshape(1, NH * CH))


# speedup vs baseline: 1.0599x; 1.0036x over previous
import functools

import jax
import jax.numpy as jnp
from jax import lax
from jax.experimental import pallas as pl
from jax.experimental.pallas import tpu as pltpu
from jax.experimental.pallas import tpu_sc as plsc

N_NODES = 10000
N_EDGES = 320000
F_IN = 16
NH = 4
CH = 128
ET = N_EDGES + N_NODES
ET_PAD = 331776
REG = 10240
GW = 128
SW = 128
BB = 2048
DB = 1000
KL = 128 // F_IN
PB = BB // KL
PROWS = ET_PAD // KL

_vector_mesh = plsc.VectorSubcoreMesh(core_axis_name="core",
                                      subcore_axis_name="subcore")
_sc_untiled = pltpu.CompilerParams(use_tc_tiling_on_sc=False)


CH_BLOCKS = (27, 27, 27, 27, 27, 27)
NCH = len(CH_BLOCKS)


def _gather_call(feat, idx_cat, ne):
    nidx = 2 * ne

    @functools.partial(
        pl.kernel,
        out_type=jax.ShapeDtypeStruct((nidx, F_IN), jnp.float32),
        mesh=_vector_mesh,
        compiler_params=_sc_untiled,
    )
    def gather_k(feat_hbm, i_hbm, o_hbm):
        def body(i_vmem, o_vmem):
            pltpu.sync_copy(feat_hbm.at[i_vmem.at[0]],
                            o_vmem.at[pl.ds(0, GW)])
            pltpu.sync_copy(feat_hbm.at[i_vmem.at[1]],
                            o_vmem.at[pl.ds(GW, GW)])

        pltpu.emit_pipeline(
            body,
            grid=(nidx // GW // 2,),
            in_specs=[pl.BlockSpec((2, GW), lambda i: (i, 0))],
            out_specs=[pl.BlockSpec((2 * GW, F_IN), lambda i: (i, 0))],
            core_axis_name=("core", "subcore"),
            dimension_semantics=(pltpu.PARALLEL,),
        )(i_hbm, o_hbm)

    return gather_k(feat, idx_cat.reshape(nidx // GW, GW)).reshape(
        2 * ne // KL, 128)


def _logit_body(gs_ref, gd_ref, wlrb_ref, b2b_ref, att2b_ref,
                r4b_ref, pmatb_ref, cvecb_ref, pay_ref):
    ps = gs_ref[...]
    pd = gd_ref[...]
    pc = jnp.concatenate([ps, pd], axis=1).astype(jnp.bfloat16)
    u = jnp.dot(pc, wlrb_ref[...], preferred_element_type=jnp.float32)
    u = u + b2b_ref[...]
    e = jnp.where(u >= 0.0, u, 0.2 * u)
    logits = jnp.dot(e.astype(jnp.bfloat16), att2b_ref[...],
                     preferred_element_type=jnp.float32)
    a = jnp.exp(logits)
    m = jnp.dot(a, r4b_ref[...], preferred_element_type=jnp.float32)
    g4 = jnp.dot(ps, pmatb_ref[...],
                 preferred_element_type=jnp.float32) + cvecb_ref[...]
    pay_ref[...] = m * g4


def _logit_call(g2, wlrb, b2b, att2b, r4b, pmatb, cvecb, ne):
    nb = ne // BB
    return pl.pallas_call(
        _logit_body,
        grid=(nb,),
        in_specs=[
            pl.BlockSpec((PB, 128), lambda i: (i, 0)),
            pl.BlockSpec((PB, 128), lambda i: (i + nb, 0)),
            pl.BlockSpec((256, KL * 512), lambda i: (0, 0)),
            pl.BlockSpec((1, KL * 512), lambda i: (0, 0)),
            pl.BlockSpec((KL * 512, KL * NH), lambda i: (0, 0)),
            pl.BlockSpec((KL * NH, KL * CH), lambda i: (0, 0)),
            pl.BlockSpec((128, KL * CH), lambda i: (0, 0)),
            pl.BlockSpec((1, KL * CH), lambda i: (0, 0)),
        ],
        out_specs=pl.BlockSpec((PB, KL * CH), lambda i: (i, 0)),
        out_shape=jax.ShapeDtypeStruct((ne // KL, KL * CH), jnp.float32),
    )(g2, g2, wlrb, b2b, att2b, r4b, pmatb, cvecb)


def _scatter_call(pay, idx_perm, zrows, ne):
    steps_r = ne // KL // SW
    slc = REG // 16

    @functools.partial(
        pl.kernel,
        out_type=jax.ShapeDtypeStruct((2, REG, CH), jnp.float32),
        mesh=_vector_mesh,
        scratch_types=[pltpu.VMEM_SHARED((REG, CH), jnp.float32)],
    )
    def scatter_k(pay_hbm, i_hbm, z_hbm, gp_hbm, table):
        c = lax.axis_index("core")
        s = lax.axis_index("subcore")
        pltpu.sync_copy(z_hbm, table.at[pl.ds(s * slc, slc)])
        plsc.subcore_barrier()

        def body(pay_vmem, i_vmem):
            pltpu.sync_copy(pay_vmem, table.at[i_vmem.at[0, 0]], add=True)

        pltpu.emit_pipeline(
            body,
            grid=(steps_r, KL),
            in_specs=[
                pl.BlockSpec((SW, 128), lambda i, k: (i, k)),
                pl.BlockSpec((1, 1, SW), lambda i, k: (i, k, 0)),
            ],
            out_specs=[],
            core_axis_name=("core", "subcore"),
            dimension_semantics=(pltpu.PARALLEL, pltpu.PARALLEL),
        )(pay_hbm, i_hbm)
        plsc.subcore_barrier()
        pltpu.sync_copy(table.at[pl.ds(s * slc, slc)],
                        gp_hbm.at[c, pl.ds(s * slc, slc)])

    return scatter_k(pay, idx_perm, zrows)


def _final_body(*refs):
    gp_refs = refs[:NCH]
    wl_ref, bl_ref, bias_ref, o_ref = refs[NCH:]
    g = gp_refs[0][0] + gp_refs[0][1]
    for r in gp_refs[1:]:
        g = g + r[0] + r[1]
    outs = []
    for h in range(NH):
        feat_sum = g[:, 32 * h:32 * h + F_IN]
        ssum = g[:, 32 * h + F_IN:32 * h + F_IN + 1]
        y = jnp.dot(feat_sum, wl_ref[:, h * CH:(h + 1) * CH],
                    preferred_element_type=jnp.float32)
        rr = 1.0 / (ssum + 1e-16)
        outs.append(y * rr + bl_ref[:, h * CH:(h + 1) * CH] * (ssum * rr)
                    + bias_ref[:, h * CH:(h + 1) * CH])
    o_ref[...] = jnp.concatenate(outs, axis=1)


def _final_call(gpairs, W_l, bl2, bias2):
    gspec = pl.BlockSpec((2, DB, CH), lambda d: (0, d, 0))
    return pl.pallas_call(
        _final_body,
        grid=(N_NODES // DB,),
        in_specs=[gspec] * NCH + [
            pl.BlockSpec((F_IN, NH * CH), lambda d: (0, 0)),
            pl.BlockSpec((1, NH * CH), lambda d: (0, 0)),
            pl.BlockSpec((1, NH * CH), lambda d: (0, 0)),
        ],
        out_specs=pl.BlockSpec((DB, NH * CH), lambda d: (d, 0)),
        out_shape=jax.ShapeDtypeStruct((N_NODES, NH * CH), jnp.float32),
    )(*gpairs, W_l, bl2, bias2)


def kernel(edge_feat, edge_to_edge_index, W_l, b_l, W_r, b_r, att, bias):
    loop = jnp.arange(N_NODES, dtype=jnp.int32)
    pad = ET_PAD - ET
    src_all = jnp.concatenate(
        [edge_to_edge_index[0], loop, jnp.zeros((pad,), jnp.int32)])
    dst_real = jnp.concatenate([edge_to_edge_index[1], loop])
    dst_g = jnp.concatenate([dst_real, jnp.zeros((pad,), jnp.int32)])
    idx_dst = jnp.concatenate(
        [dst_real, jnp.full((pad,), N_NODES, jnp.int32)])
    idx_perm = idx_dst.reshape(PROWS // SW, SW, KL).transpose(0, 2, 1)
    b2 = (b_l + b_r).reshape(1, NH * CH)
    eye4 = jnp.eye(NH, dtype=jnp.float32)
    eye8 = jnp.eye(KL, dtype=jnp.float32)
    att2 = (eye4[:, None, :] * att[:, :, None]).reshape(NH * CH, NH)
    lane = jnp.arange(CH)
    r4 = (eye4[:, lane // 32] * (lane % 32 < 17)[None, :]).astype(jnp.float32)
    pmat = ((lane % 32)[None, :] == jnp.arange(F_IN)[:, None]).astype(
        jnp.float32)
    cvec = ((lane % 32) == F_IN).astype(jnp.float32).reshape(1, CH)
    wlrb = jnp.concatenate(
        [jnp.kron(eye8, W_l), jnp.kron(eye8, W_r)], axis=0
    ).astype(jnp.bfloat16)
    b2b = jnp.tile(b2, (1, KL))
    att2b = jnp.kron(eye8, att2).astype(jnp.bfloat16)
    r4b = jnp.kron(eye8, r4)
    pmatb = jnp.kron(eye8, pmat)
    cvecb = jnp.tile(cvec, (1, KL))
    zrows = jnp.zeros((REG // 16, CH), jnp.float32)

    gpairs = []
    e0 = 0
    for c in range(NCH):
        ne = CH_BLOCKS[c] * BB
        idx_cat_c = jnp.concatenate([src_all[e0:e0 + ne],
                                     dst_g[e0:e0 + ne]])
        g2c = _gather_call(edge_feat, idx_cat_c, ne)
        payc = _logit_call(g2c, wlrb, b2b, att2b, r4b, pmatb, cvecb, ne)
        s0 = e0 // KL // SW
        gpairs.append(_scatter_call(
            payc, idx_perm[s0:s0 + ne // KL // SW], zrows, ne))
        e0 += ne
    return _final_call(gpairs, W_l, b_l.reshape(1, NH * CH),
                       bias.reshape(1, NH * CH))
